# Initial kernel scaffold; baseline (speedup 1.0000x reference)
import jax
import jax.numpy as jnp
from jax.experimental import pallas as pl


def kernel(static_entity_embed, static_relation_embed, gate_weight, gate_bias, gru_W_ih, gru_W_hh, gru_b_ih, gru_b_hh, rgcn_Wn, rgcn_Ws, edges):
    raise NotImplementedError("write your pallas kernel here")


# speedup vs baseline: 3.4450x; 3.4450x over previous
import functools

import jax
import jax.numpy as jnp
from jax import lax
from jax.experimental import pallas as pl
from jax.experimental.pallas import tpu as pltpu
from jax.experimental.pallas import tpu_sc as plsc

NC = 2
NS = 16
LN = 16
NW = NC * NS
CH = 80

F32 = jnp.float32
I32 = jnp.int32


def _mesh():
    return plsc.VectorSubcoreMesh(core_axis_name="c", subcore_axis_name="s")


def _zero_rows(rows, h):
    def zrow(i, c):
        for c1 in range(h // LN):
            rows[i, pl.ds(c1 * LN, LN)] = jnp.zeros((LN,), F32)
        return c
    lax.fori_loop(0, rows.shape[0], zrow, 0)


def _zero_hist(buf):
    def zrow(i, c):
        for c1 in range(buf.shape[1] // LN):
            buf[i, pl.ds(c1 * LN, LN)] = jnp.zeros((LN,), F32)
        return c
    lax.fori_loop(0, buf.shape[0], zrow, 0)


def _fill_identity(idbuf):
    for v in range(idbuf.shape[1] // LN):
        idbuf[0, pl.ds(v * LN, LN)] = v * LN + jnp.arange(LN, dtype=I32)


def _zero_acc(rows, acc, sid, rpt):
    nb = rows.shape[0]
    for k in range(rpt // nb):
        pltpu.sync_copy(rows, acc.at[pl.ds(sid * rpt + k * nb, nb)])


def _writeout(acc, out, cid, sid, rpt):
    pltpu.sync_copy(acc.at[pl.ds(sid * rpt, rpt)],
                    out.at[cid, pl.ds(sid * rpt, rpt)])


def _make_segsum(n_tab, n_edges, npad, h):
    per_w = n_edges // NW
    assert n_edges % NW == 0 and per_w % CH == 0
    rpt = npad // NS

    @functools.partial(
        pl.kernel,
        out_type=jax.ShapeDtypeStruct((NC, npad, h), F32),
        mesh=_mesh(),
        compiler_params=pltpu.CompilerParams(needs_layout_passes=False),
        scratch_types=[
            pltpu.VMEM((CH, h), F32),
            pltpu.VMEM((CH,), I32),
            pltpu.VMEM((1, CH), I32),
            pltpu.VMEM_SHARED((npad, h), F32),
        ],
    )
    def k(table_hbm, gidx_hbm, sidx_hbm, out_hbm, rows, gbuf, sb2, acc):
        cid = lax.axis_index("c")
        sid = lax.axis_index("s")
        wid = cid * NS + sid

        _zero_rows(rows, h)
        _zero_acc(rows, acc, sid, rpt)
        plsc.subcore_barrier()

        def body(j, c):
            base = wid * per_w + j * CH
            pltpu.sync_copy(gidx_hbm.at[pl.ds(base, CH)], gbuf)
            pltpu.sync_copy(sidx_hbm.at[pl.ds(base, CH)], sb2.at[0])
            pltpu.sync_copy(table_hbm.at[gbuf], rows)
            pltpu.sync_copy(rows, acc.at[sb2.at[0]], add=True)
            return c
        lax.fori_loop(0, per_w // CH, body, 0)

        plsc.subcore_barrier()
        _writeout(acc, out_hbm, cid, sid, rpt)

    return k


def _make_pass1(n_ent, h, n_edges, npad, m, msz):
    per_w = n_edges // NW
    per_w2 = (2 * n_edges) // NW
    assert per_w % CH == 0 and per_w2 % CH == 0
    rpt = npad // NS
    hr = npad // 128
    hrt = hr // NS

    @functools.partial(
        pl.kernel,
        out_type=(
            jax.ShapeDtypeStruct((NC, npad, h), F32),
            jax.ShapeDtypeStruct((NC, hr, 128), F32),
            jax.ShapeDtypeStruct((msz,), I32),
        ),
        mesh=_mesh(),
        compiler_params=pltpu.CompilerParams(needs_layout_passes=False),
        scratch_types=[
            pltpu.VMEM((CH, h), F32),
            pltpu.VMEM((CH,), I32),
            pltpu.VMEM((1, CH), I32),
            pltpu.VMEM((CH,), I32),
            pltpu.VMEM((CH,), I32),
            pltpu.VMEM((1, CH), I32),
            pltpu.VMEM((CH,), I32),
            pltpu.VMEM((hr, 128), F32),
            pltpu.VMEM((1, hr), I32),
            pltpu.VMEM_SHARED((npad, h), F32),
            pltpu.VMEM_SHARED((hr, 128), F32),
        ],
    )
    def k(h_hbm, src_hbm, dst_hbm, ent_hbm, rel_hbm,
          a_out, d_out, mk_out,
          rows, gbuf, sb2, ebuf, rbuf, cb2, vbuf, degloc, idbuf, acc, dacc):
        cid = lax.axis_index("c")
        sid = lax.axis_index("s")
        wid = cid * NS + sid
        ones16 = jnp.ones((LN,), F32)

        _zero_rows(rows, h)
        _zero_hist(degloc)
        _fill_identity(idbuf)
        _zero_acc(rows, acc, sid, rpt)
        @pl.when(sid < hr // 8)
        def _():
            pltpu.sync_copy(degloc.at[pl.ds(sid * 8, 8)],
                            dacc.at[pl.ds(sid * 8, 8)])
        plsc.subcore_barrier()

        def body_a(j, c):
            base = wid * per_w + j * CH
            pltpu.sync_copy(src_hbm.at[pl.ds(base, CH)], gbuf)
            pltpu.sync_copy(dst_hbm.at[pl.ds(base, CH)], sb2.at[0])
            pltpu.sync_copy(h_hbm.at[gbuf], rows)
            pltpu.sync_copy(rows, acc.at[sb2.at[0]], add=True)
            for v in range(CH // LN):
                didx = sb2[0, pl.ds(v * LN, LN)]
                plsc.addupdate_scatter(degloc, [lax.shift_right_logical(didx, 7),
                                                lax.bitwise_and(didx, 127)], ones16)
            return c
        lax.fori_loop(0, per_w // CH, body_a, 0)

        def body_m(j, c):
            base = wid * per_w2 + j * CH
            pltpu.sync_copy(ent_hbm.at[pl.ds(base, CH)], ebuf)
            pltpu.sync_copy(rel_hbm.at[pl.ds(base, CH)], rbuf)
            for v in range(CH // LN):
                sl = pl.ds(v * LN, LN)
                cb2[0, sl] = ebuf[sl] * m + rbuf[sl]
                vbuf[sl] = base + v * LN + jnp.arange(LN, dtype=I32)
            pltpu.sync_copy(vbuf, mk_out.at[cb2.at[0]])
            return c
        lax.fori_loop(0, per_w2 // CH, body_m, 0)

        pltpu.sync_copy(degloc, dacc.at[idbuf.at[0]], add=True)
        plsc.subcore_barrier()
        _writeout(acc, a_out, cid, sid, rpt)
        @pl.when(sid < hr // 8)
        def _():
            pltpu.sync_copy(dacc.at[pl.ds(sid * 8, 8)],
                            d_out.at[cid, pl.ds(sid * 8, 8)])

    return k


def _make_pass2(n_ent, h, n_edges, npad, m, msz, dummy):
    per_w2 = (2 * n_edges) // NW
    assert per_w2 % CH == 0
    rpt = npad // NS
    hr = npad // 128
    hrt = hr // NS

    @functools.partial(
        pl.kernel,
        out_type=(
            jax.ShapeDtypeStruct((NC, npad, h), F32),
            jax.ShapeDtypeStruct((NC, hr, 128), F32),
        ),
        mesh=_mesh(),
        compiler_params=pltpu.CompilerParams(needs_layout_passes=False),
        scratch_types=[
            pltpu.VMEM((CH, h), F32),
            pltpu.VMEM((CH,), I32),
            pltpu.VMEM((CH,), I32),
            pltpu.VMEM((1, CH), I32),
            pltpu.VMEM((CH,), I32),
            pltpu.VMEM((1, CH), I32),
            pltpu.VMEM((hr, 128), F32),
            pltpu.VMEM((1, hr), I32),
            pltpu.VMEM_SHARED((npad, h), F32),
            pltpu.VMEM_SHARED((hr, 128), F32),
        ],
    )
    def k(h_hbm, ent_hbm, rel_hbm, mk_hbm,
          s_out, c_out,
          rows, ebuf, rbuf, cb2, mbuf, sb2, cntloc, idbuf, acc, cacc):
        cid = lax.axis_index("c")
        sid = lax.axis_index("s")
        wid = cid * NS + sid
        ones16 = jnp.ones((LN,), F32)

        _zero_rows(rows, h)
        _zero_hist(cntloc)
        _fill_identity(idbuf)
        _zero_acc(rows, acc, sid, rpt)
        @pl.when(sid < hr // 8)
        def _():
            pltpu.sync_copy(cntloc.at[pl.ds(sid * 8, 8)],
                            cacc.at[pl.ds(sid * 8, 8)])
        plsc.subcore_barrier()

        def body(j, c):
            base = wid * per_w2 + j * CH
            pltpu.sync_copy(ent_hbm.at[pl.ds(base, CH)], ebuf)
            pltpu.sync_copy(rel_hbm.at[pl.ds(base, CH)], rbuf)
            for v in range(CH // LN):
                sl = pl.ds(v * LN, LN)
                cb2[0, sl] = ebuf[sl] * m + rbuf[sl]
            pltpu.sync_copy(mk_hbm.at[cb2.at[0]], mbuf)
            for v in range(CH // LN):
                sl = pl.ds(v * LN, LN)
                gid = base + v * LN + jnp.arange(LN, dtype=I32)
                marked = mbuf[sl] == gid
                rv = rbuf[sl]
                sb2[0, sl] = jnp.where(marked, rv, jnp.full((LN,), dummy, I32))
                plsc.addupdate_scatter(cntloc, [lax.shift_right_logical(rv, 7),
                                                lax.bitwise_and(rv, 127)], ones16,
                                       mask=marked)
            pltpu.sync_copy(h_hbm.at[ebuf], rows)
            pltpu.sync_copy(rows, acc.at[sb2.at[0]], add=True)
            return c
        lax.fori_loop(0, per_w2 // CH, body, 0)

        pltpu.sync_copy(cntloc, cacc.at[idbuf.at[0]], add=True)
        plsc.subcore_barrier()
        _writeout(acc, s_out, cid, sid, rpt)
        @pl.when(sid < hr // 8)
        def _():
            pltpu.sync_copy(cacc.at[pl.ds(sid * 8, 8)],
                            c_out.at[cid, pl.ds(sid * 8, 8)])

    return k


RB = 1024


def _gates(gi, gh, hprev, h):
    r = jax.nn.sigmoid(gi[:, :h] + gh[:, :h])
    z = jax.nn.sigmoid(gi[:, h:2 * h] + gh[:, h:2 * h])
    n = jnp.tanh(gi[:, 2 * h:] + r * gh[:, 2 * h:])
    return (1.0 - z) * n + z * hprev


def _grid(rows):
    return ((rows + RB - 1) // RB,)


def _tc_norm(x):
    e, h = x.shape

    def body(x_ref, o_ref):
        v = x_ref[...]
        nrm = jnp.sqrt(jnp.sum(v * v, axis=1, keepdims=True))
        o_ref[...] = v / jnp.maximum(nrm, 1e-12)

    return pl.pallas_call(
        body,
        grid=_grid(e),
        in_specs=[pl.BlockSpec((RB, h), lambda i: (i, 0))],
        out_specs=pl.BlockSpec((RB, h), lambda i: (i, 0)),
        out_shape=jax.ShapeDtypeStruct((e, h), F32),
    )(x)


def _tc_gru0(srel, wih_s, whh_t, bih, bhh):
    mrows, h = srel.shape

    def body(s_ref, ws_ref, wh_ref, bi_ref, bh_ref, o_ref):
        s = s_ref[...]
        gi = jnp.dot(s, ws_ref[...], preferred_element_type=F32) + bi_ref[...]
        gh = jnp.dot(s, wh_ref[...], preferred_element_type=F32) + bh_ref[...]
        o_ref[...] = _gates(gi, gh, s, h)

    return pl.pallas_call(
        body,
        grid=_grid(mrows),
        in_specs=[
            pl.BlockSpec((RB, h), lambda i: (i, 0)),
            pl.BlockSpec(wih_s.shape, lambda i: (0, 0)),
            pl.BlockSpec(whh_t.shape, lambda i: (0, 0)),
            pl.BlockSpec(bih.shape, lambda i: (0, 0)),
            pl.BlockSpec(bhh.shape, lambda i: (0, 0)),
        ],
        out_specs=pl.BlockSpec((RB, h), lambda i: (i, 0)),
        out_shape=jax.ShapeDtypeStruct((mrows, h), F32),
    )(srel, wih_s, whh_t, bih, bhh)


def _tc_gru(rsum_p, cnt_p, srel, hrel, wih_s, wih_c, whh_t, bih, bhh):
    mrows, h = srel.shape
    npad = rsum_p.shape[1]

    def body(rs_ref, cn_ref, s_ref, h_ref, ws_ref, wc_ref, wh_ref, bi_ref, bh_ref, o_ref):
        pid = pl.program_id(0)
        s = rs_ref[0] + rs_ref[1]
        off = pl.multiple_of(pid * RB, 128)
        cnt = cn_ref[0, pl.ds(off, RB)] + cn_ref[1, pl.ds(off, RB)]
        cur = s / jnp.maximum(cnt, 1.0)[:, None]
        st = s_ref[...]
        hp = h_ref[...]
        gi = (jnp.dot(st, ws_ref[...], preferred_element_type=F32)
              + jnp.dot(cur, wc_ref[...], preferred_element_type=F32) + bi_ref[...])
        gh = jnp.dot(hp, wh_ref[...], preferred_element_type=F32) + bh_ref[...]
        o_ref[...] = _gates(gi, gh, hp, h)

    return pl.pallas_call(
        body,
        grid=_grid(mrows),
        in_specs=[
            pl.BlockSpec((NC, RB, h), lambda i: (0, i, 0)),
            pl.BlockSpec((NC, npad), lambda i: (0, 0)),
            pl.BlockSpec((RB, h), lambda i: (i, 0)),
            pl.BlockSpec((RB, h), lambda i: (i, 0)),
            pl.BlockSpec(wih_s.shape, lambda i: (0, 0)),
            pl.BlockSpec(wih_c.shape, lambda i: (0, 0)),
            pl.BlockSpec(whh_t.shape, lambda i: (0, 0)),
            pl.BlockSpec(bih.shape, lambda i: (0, 0)),
            pl.BlockSpec(bhh.shape, lambda i: (0, 0)),
        ],
        out_specs=pl.BlockSpec((RB, h), lambda i: (i, 0)),
        out_shape=jax.ShapeDtypeStruct((mrows, h), F32),
    )(rsum_p, cnt_p, srel, hrel, wih_s, wih_c, whh_t, bih, bhh)


def _tc_layer(a_p, b_p, deg_p, hcur, wn, ws):
    e, h = hcur.shape
    npad = a_p.shape[1]

    def body(a_ref, b_ref, d_ref, h_ref, wn_ref, ws_ref, o_ref):
        pid = pl.program_id(0)
        s = a_ref[0] + a_ref[1] + b_ref[0] + b_ref[1]
        off = pl.multiple_of(pid * RB, 128)
        deg = d_ref[0, pl.ds(off, RB)] + d_ref[1, pl.ds(off, RB)]
        agg = s / jnp.maximum(deg, 1.0)[:, None]
        o_ref[...] = (jnp.dot(agg, wn_ref[...], preferred_element_type=F32)
                      + jnp.dot(h_ref[...], ws_ref[...], preferred_element_type=F32))

    return pl.pallas_call(
        body,
        grid=_grid(e),
        in_specs=[
            pl.BlockSpec((NC, RB, h), lambda i: (0, i, 0)),
            pl.BlockSpec((NC, RB, h), lambda i: (0, i, 0)),
            pl.BlockSpec((NC, npad), lambda i: (0, 0)),
            pl.BlockSpec((RB, h), lambda i: (i, 0)),
            pl.BlockSpec((h, h), lambda i: (0, 0)),
            pl.BlockSpec((h, h), lambda i: (0, 0)),
        ],
        out_specs=pl.BlockSpec((RB, h), lambda i: (i, 0)),
        out_shape=jax.ShapeDtypeStruct((e, h), F32),
    )(a_p, b_p, deg_p, hcur, wn, ws)


def _tc_gate(h0, hfin, gw, gb):
    e, h = h0.shape

    def body(h0_ref, hf_ref, gw_ref, gb_ref, o_ref):
        h0v = h0_ref[...]
        g = jax.nn.sigmoid(jnp.dot(h0v, gw_ref[...], preferred_element_type=F32)
                           + gb_ref[...])
        o_ref[...] = g * hf_ref[...] + (1.0 - g) * h0v

    return pl.pallas_call(
        body,
        grid=_grid(e),
        in_specs=[
            pl.BlockSpec((RB, h), lambda i: (i, 0)),
            pl.BlockSpec((RB, h), lambda i: (i, 0)),
            pl.BlockSpec((h, h), lambda i: (0, 0)),
            pl.BlockSpec(gb.shape, lambda i: (0, 0)),
        ],
        out_specs=pl.BlockSpec((RB, h), lambda i: (i, 0)),
        out_shape=jax.ShapeDtypeStruct((e, h), F32),
    )(h0, hfin, gw, gb)


def kernel(static_entity_embed, static_relation_embed, gate_weight, gate_bias,
           gru_W_ih, gru_W_hh, gru_b_ih, gru_b_hh, rgcn_Wn, rgcn_Ws, edges):
    e, h = static_entity_embed.shape
    m = static_relation_embed.shape[0]
    t_steps, n_edges, _ = edges.shape
    n_layers = rgcn_Wn.shape[0]
    msz = e * m
    npad = ((max(e, m) + NS * 128 - 1) // (NS * 128)) * (NS * 128)
    dummy = m

    wih_t = gru_W_ih.T
    wih_s = wih_t[:h]
    wih_c = wih_t[h:]
    whh_t = gru_W_hh.T
    bih = gru_b_ih[None, :]
    bhh = gru_b_hh[None, :]
    gb = gate_bias[None, :]

    pass1 = _make_pass1(e, h, n_edges, npad, m, msz)
    pass2 = _make_pass2(e, h, n_edges, npad, m, msz, dummy)
    segsum_rel = _make_segsum(m, n_edges, npad, h)
    segsum_ent = _make_segsum(e, n_edges, npad, h)

    hent = _tc_norm(static_entity_embed)
    rel = _tc_gru0(static_relation_embed, wih_s, whh_t, bih, bhh)

    for t in range(t_steps):
        src = edges[t, :, 0]
        r = edges[t, :, 1]
        dst = edges[t, :, 2]
        ent = jnp.concatenate([src, dst])
        rel_of_pair = jnp.concatenate([r, r])

        a0_p, deg_p, marker = pass1(hent, src, dst, ent, rel_of_pair)
        deg_p = deg_p.reshape(NC, npad)
        rsum_p, cnt_p = pass2(hent, ent, rel_of_pair, marker)
        cnt_p = cnt_p.reshape(NC, npad)
        rel = _tc_gru(rsum_p, cnt_p, static_relation_embed, rel,
                      wih_s, wih_c, whh_t, bih, bhh)
        b_p = segsum_rel(rel, r, dst)

        h0 = hent
        hcur = h0
        for l in range(n_layers):
            a_p = a0_p if l == 0 else segsum_ent(hcur, src, dst)
            hcur = _tc_layer(a_p, b_p, deg_p, hcur, rgcn_Wn[l], rgcn_Ws[l])
        hent = _tc_gate(h0, hcur, gate_weight, gb)

    return (hent, rel)
